# baseline (speedup 1.0000x reference)
import functools

import jax
import jax.numpy as jnp
from jax import lax
from jax.experimental import pallas as pl
from jax.experimental.pallas import tpu as pltpu
from jax.experimental.pallas import tpu_sc as plsc

_DIM = 1024
_SEQ = 4096
_SCALE = _DIM ** (-0.5)
_TC_ROWS = 2048
_SC_ROWS = _SEQ - _TC_ROWS
_NC, _NS, _L = 2, 16, 16
_NW = _NC * _NS
_ROWS_W = _SC_ROWS // _NW
_CROWS = 8
_NCHUNK = _ROWS_W // _CROWS
_NBUF = 4
_UNROLL = 16
_NVEC = _DIM // (_L * _UNROLL)

_mesh = plsc.VectorSubcoreMesh(core_axis_name="c", subcore_axis_name="s")


@functools.partial(
    pl.kernel,
    mesh=_mesh,
    out_type=jax.ShapeDtypeStruct((_SC_ROWS, _DIM), jnp.float32),
    scratch_types=(
        [pltpu.VMEM((_CROWS, _DIM), jnp.float32) for _ in range(2 * _NBUF)]
        + [pltpu.SemaphoreType.DMA for _ in range(2 * _NBUF)]
    ),
)
def _sc_scale_copy(emb_hbm, out_hbm, *scratch):
    wid = lax.axis_index("s") * _NC + lax.axis_index("c")
    base = wid * _ROWS_W
    ibufs = scratch[:_NBUF]
    obufs = scratch[_NBUF:2 * _NBUF]
    isems = scratch[2 * _NBUF:3 * _NBUF]
    osems = scratch[3 * _NBUF:]

    def in_copy(c, p):
        r0 = _TC_ROWS + base + c * _CROWS
        return pltpu.make_async_copy(
            emb_hbm.at[pl.ds(r0, _CROWS)], ibufs[p], isems[p])

    def out_copy(c, p):
        dst = out_hbm.at[pl.ds(base + c * _CROWS, _CROWS)]
        return pltpu.make_async_copy(obufs[p], dst, osems[p])

    def compute(p):
        src, dst = ibufs[p], obufs[p]

        def row(r, outer):
            srow, drow = src.at[r], dst.at[r]

            def vec(j, inner):
                b = j * (_L * _UNROLL)
                for u in range(_UNROLL):
                    sl = pl.ds(b + u * _L, _L)
                    drow[sl] = srow[sl] * _SCALE
                return inner

            lax.fori_loop(0, _NVEC, vec, 0)
            return outer

        lax.fori_loop(0, _CROWS, row, 0)

    for c in range(_NBUF):
        in_copy(c, c).start()

    _NGRP = _NCHUNK // _NBUF

    def grp(g, carry):
        for p in range(_NBUF):
            c = g * _NBUF + p

            @pl.when(g >= 1)
            def _():
                out_copy(c - _NBUF, p).wait()

            in_copy(c, p).wait()
            compute(p)
            out_copy(c, p).start()

            @pl.when(g < _NGRP - 1)
            def _():
                in_copy(c + _NBUF, p).start()

        return carry

    lax.fori_loop(0, _NGRP, grp, 0)

    for c in range(_NCHUNK - _NBUF, _NCHUNK):
        out_copy(c, c % _NBUF).wait()


_TC_BLOCK = 256


def _tc_body(emb_ref, out_ref):
    out_ref[...] = emb_ref[...] * _SCALE


_tc_scale = pl.pallas_call(
    _tc_body,
    grid=(_TC_ROWS // _TC_BLOCK,),
    in_specs=[pl.BlockSpec((_TC_BLOCK, _DIM), lambda i: (i, 0))],
    out_specs=pl.BlockSpec((_TC_BLOCK, _DIM), lambda i: (i, 0)),
    out_shape=jax.ShapeDtypeStruct((_TC_ROWS, _DIM), jnp.float32),
)


def kernel(x, emb):
    del x
    sc_part = _sc_scale_copy(emb)
    tc_part = _tc_scale(emb)
    return jnp.concatenate([tc_part, sc_part], axis=0)


# speedup vs baseline: 1.2605x; 1.2605x over previous
import functools

import jax
import jax.numpy as jnp
from jax import lax
from jax.experimental import pallas as pl
from jax.experimental.pallas import tpu as pltpu
from jax.experimental.pallas import tpu_sc as plsc

_DIM = 1024
_SEQ = 4096
_SCALE = _DIM ** (-0.5)
_NC, _NS, _L = 2, 16, 16
_NW = _NC * _NS
_ROWS_W = _SEQ // _NW
_CROWS = 16
_NCHUNK = _ROWS_W // _CROWS
_UNROLL = 16
_NVEC = _DIM // (_L * _UNROLL)

_mesh = plsc.VectorSubcoreMesh(core_axis_name="c", subcore_axis_name="s")


@functools.partial(
    pl.kernel,
    mesh=_mesh,
    out_type=jax.ShapeDtypeStruct((_SEQ, _DIM), jnp.float32),
    scratch_types=[
        pltpu.VMEM((_CROWS, _DIM), jnp.float32),
        pltpu.VMEM((_CROWS, _DIM), jnp.float32),
        pltpu.VMEM((_CROWS, _DIM), jnp.float32),
        pltpu.VMEM((_CROWS, _DIM), jnp.float32),
        pltpu.SemaphoreType.DMA,
        pltpu.SemaphoreType.DMA,
        pltpu.SemaphoreType.DMA,
        pltpu.SemaphoreType.DMA,
    ],
)
def _sc_scale_copy(emb_hbm, out_hbm, ibuf0, ibuf1, obuf0, obuf1,
                   isem0, isem1, osem0, osem1):
    wid = lax.axis_index("s") * _NC + lax.axis_index("c")
    base = wid * _ROWS_W
    ibufs, obufs = (ibuf0, ibuf1), (obuf0, obuf1)
    isems, osems = (isem0, isem1), (osem0, osem1)

    def in_copy(c, p):
        src = emb_hbm.at[pl.ds(base + c * _CROWS, _CROWS)]
        return pltpu.make_async_copy(src, ibufs[p], isems[p])

    def out_copy(c, p):
        dst = out_hbm.at[pl.ds(base + c * _CROWS, _CROWS)]
        return pltpu.make_async_copy(obufs[p], dst, osems[p])

    def compute(p):
        src, dst = ibufs[p], obufs[p]

        def row(r, outer):
            srow, drow = src.at[r], dst.at[r]

            def vec(j, inner):
                b = j * (_L * _UNROLL)
                for u in range(_UNROLL):
                    sl = pl.ds(b + u * _L, _L)
                    drow[sl] = srow[sl] * _SCALE
                return inner

            lax.fori_loop(0, _NVEC, vec, 0)
            return outer

        lax.fori_loop(0, _CROWS, row, 0)

    in_copy(0, 0).start()
    in_copy(1, 1).start()

    _NPAIR = _NCHUNK // 2

    def pair(g, carry):
        for p in (0, 1):
            c = g * 2 + p

            @pl.when(g >= 1)
            def _():
                out_copy(c - 2, p).wait()

            in_copy(c, p).wait()
            compute(p)
            out_copy(c, p).start()

            @pl.when(g < _NPAIR - 1)
            def _():
                in_copy(c + 2, p).start()

        return carry

    lax.fori_loop(0, _NPAIR, pair, 0)

    out_copy(_NCHUNK - 2, 0).wait()
    out_copy(_NCHUNK - 1, 1).wait()


def kernel(x, emb):
    del x
    return _sc_scale_copy(emb)
